# Initial kernel scaffold; baseline (speedup 1.0000x reference)
import jax
import jax.numpy as jnp
from jax.experimental import pallas as pl


def kernel(token_ids, embedding_matrix):
    raise NotImplementedError("write your pallas kernel here")


# speedup vs baseline: 1.0944x; 1.0944x over previous
import functools

import jax
import jax.numpy as jnp
from jax import lax
from jax.experimental import pallas as pl
from jax.experimental.pallas import tpu as pltpu
from jax.experimental.pallas import tpu_sc as plsc

NUM_CORES = 2
NUM_SUBCORES = 16
NUM_WORKERS = NUM_CORES * NUM_SUBCORES

GATHER_ROWS = 128
CHUNKS = 8
BLOCK_ROWS = GATHER_ROWS * CHUNKS


def _embed_lookup(ids_hbm, table_hbm, out_hbm, idx_v, rows_v, sem):
    wid = lax.axis_index("s") * NUM_CORES + lax.axis_index("c")
    n_blocks_total = ids_hbm.shape[0]
    n_blocks_w = n_blocks_total // NUM_WORKERS

    @pl.loop(0, n_blocks_w)
    def _block(i):
        blk = wid * n_blocks_w + i
        base = blk * BLOCK_ROWS
        pltpu.sync_copy(ids_hbm.at[blk], idx_v)
        copies = []
        for c in range(CHUNKS):
            copies.append(
                pltpu.async_copy(
                    table_hbm.at[idx_v.at[c]],
                    rows_v.at[pl.ds(c * GATHER_ROWS, GATHER_ROWS)],
                    sem,
                )
            )
        for cp in copies:
            cp.wait()
        pltpu.sync_copy(rows_v, out_hbm.at[pl.ds(base, BLOCK_ROWS)])


def kernel(token_ids, embedding_matrix):
    batch, seq = token_ids.shape
    n_rows, dim = embedding_matrix.shape
    total = batch * seq
    assert total % (NUM_WORKERS * BLOCK_ROWS) == 0
    flat_ids = token_ids.reshape(total // BLOCK_ROWS, CHUNKS, GATHER_ROWS)
    flat_ids = flat_ids.astype(jnp.int32)

    mesh = plsc.VectorSubcoreMesh(
        core_axis_name="c", subcore_axis_name="s",
        num_cores=NUM_CORES, num_subcores=NUM_SUBCORES,
    )
    run = pl.kernel(
        _embed_lookup,
        out_type=jax.ShapeDtypeStruct((total, dim), jnp.float32),
        mesh=mesh,
        scratch_types=[
            pltpu.VMEM((CHUNKS, GATHER_ROWS), jnp.int32),
            pltpu.VMEM((BLOCK_ROWS, dim), jnp.float32),
            pltpu.SemaphoreType.DMA,
        ],
        compiler_params=pltpu.CompilerParams(use_tc_tiling_on_sc=False),
    )
    out = run(flat_ids, embedding_matrix)
    return out.reshape(batch, seq, dim)


# speedup vs baseline: 1.1102x; 1.0144x over previous
import jax
import jax.numpy as jnp
from jax import lax
from jax.experimental import pallas as pl
from jax.experimental.pallas import tpu as pltpu
from jax.experimental.pallas import tpu_sc as plsc

NUM_CORES = 2
NUM_SUBCORES = 16
NUM_WORKERS = NUM_CORES * NUM_SUBCORES

GATHER_ROWS = 128
CHUNKS = 10
BLOCK_ROWS = GATHER_ROWS * CHUNKS


def _embed_lookup(ids_hbm, table_hbm, out_hbm, idx_v, rows_v,
                  sem_g, sem_i0, sem_i1, sem_s0, sem_s1):
    wid = lax.axis_index("s") * NUM_CORES + lax.axis_index("c")
    n_blocks_total = ids_hbm.shape[0]
    nbw = n_blocks_total // NUM_WORKERS
    base_blk = wid * nbw
    sem_i = (sem_i0, sem_i1)
    sem_s = (sem_s0, sem_s1)

    pltpu.async_copy(ids_hbm.at[base_blk], idx_v.at[0], sem_i[0])

    @pl.loop(0, nbw, step=2)
    def _pair(i0):
        for b in range(2):
            i = i0 + b
            blk = base_blk + i

            pltpu.make_async_copy(ids_hbm.at[blk], idx_v.at[b], sem_i[b]).wait()

            @pl.when(i >= 2)
            def _():
                pltpu.make_async_copy(
                    rows_v.at[b], out_hbm.at[pl.ds(0, BLOCK_ROWS)], sem_s[b]
                ).wait()

            copies = [
                pltpu.async_copy(
                    table_hbm.at[idx_v.at[b, c]],
                    rows_v.at[b, pl.ds(c * GATHER_ROWS, GATHER_ROWS)],
                    sem_g,
                )
                for c in range(CHUNKS)
            ]

            @pl.when(i + 1 < nbw)
            def _():
                pltpu.async_copy(
                    ids_hbm.at[blk + 1], idx_v.at[1 - b], sem_i[1 - b]
                )

            for cp in copies:
                cp.wait()

            pltpu.async_copy(
                rows_v.at[b],
                out_hbm.at[pl.ds(blk * BLOCK_ROWS, BLOCK_ROWS)],
                sem_s[b],
            )

    for b in range(2):
        pltpu.make_async_copy(
            rows_v.at[b], out_hbm.at[pl.ds(0, BLOCK_ROWS)], sem_s[b]
        ).wait()


def kernel(token_ids, embedding_matrix):
    batch, seq = token_ids.shape
    n_rows, dim = embedding_matrix.shape
    total = batch * seq
    assert total % (NUM_WORKERS * BLOCK_ROWS) == 0
    assert (total // (NUM_WORKERS * BLOCK_ROWS)) % 2 == 0
    flat_ids = token_ids.reshape(total // BLOCK_ROWS, CHUNKS, GATHER_ROWS)
    flat_ids = flat_ids.astype(jnp.int32)

    mesh = plsc.VectorSubcoreMesh(
        core_axis_name="c", subcore_axis_name="s",
        num_cores=NUM_CORES, num_subcores=NUM_SUBCORES,
    )
    run = pl.kernel(
        _embed_lookup,
        out_type=jax.ShapeDtypeStruct((total, dim), jnp.float32),
        mesh=mesh,
        scratch_types=[
            pltpu.VMEM((2, CHUNKS, GATHER_ROWS), jnp.int32),
            pltpu.VMEM((2, BLOCK_ROWS, dim), jnp.float32),
            pltpu.SemaphoreType.DMA,
            pltpu.SemaphoreType.DMA,
            pltpu.SemaphoreType.DMA,
            pltpu.SemaphoreType.DMA,
            pltpu.SemaphoreType.DMA,
        ],
        compiler_params=pltpu.CompilerParams(use_tc_tiling_on_sc=False),
    )
    out = run(flat_ids, embedding_matrix)
    return out.reshape(batch, seq, dim)


# speedup vs baseline: 1.7918x; 1.6140x over previous
import jax
import jax.numpy as jnp
from jax import lax
from jax.experimental import pallas as pl
from jax.experimental.pallas import tpu as pltpu
from jax.experimental.pallas import tpu_sc as plsc

NUM_CORES = 2
NUM_SUBCORES = 16
NUM_WORKERS = NUM_CORES * NUM_SUBCORES

R = 16
SEQ = 50


def _embed_lookup(ids_hbm, table_hbm, out_hbm, idx_v, rows_v,
                  sem_i0, sem_i1, sem_g, sem_s0, sem_s1):
    wid = lax.axis_index("s") * NUM_CORES + lax.axis_index("c")
    n_rows_total = ids_hbm.shape[0]
    nbw = n_rows_total // (NUM_WORKERS * R)
    base_blk = wid * nbw
    sem_i = (sem_i0, sem_i1)
    sem_s = (sem_s0, sem_s1)

    def load_ids(blk, b, sem):
        pltpu.async_copy(
            ids_hbm.at[pl.ds(blk * R, R)], idx_v.at[b], sem
        )

    load_ids(base_blk, 0, sem_i[0])

    @pl.loop(0, nbw, step=2)
    def _pair(i0):
        for b in range(2):
            i = i0 + b
            blk = base_blk + i

            pltpu.make_async_copy(
                ids_hbm.at[pl.ds(0, R)], idx_v.at[b], sem_i[b]
            ).wait()

            @pl.when(i >= 2)
            def _():
                pltpu.make_async_copy(
                    rows_v.at[b], out_hbm.at[pl.ds(0, R)], sem_s[b]
                ).wait()

            copies = [
                pltpu.async_copy(
                    table_hbm.at[idx_v.at[b, r]],
                    rows_v.at[b, r],
                    sem_g,
                )
                for r in range(R)
            ]

            @pl.when(i + 1 < nbw)
            def _():
                load_ids(blk + 1, 1 - b, sem_i[1 - b])

            for cp in copies:
                cp.wait()

            pltpu.async_copy(
                rows_v.at[b], out_hbm.at[pl.ds(blk * R, R)], sem_s[b]
            )

    for b in range(2):
        pltpu.make_async_copy(
            rows_v.at[b], out_hbm.at[pl.ds(0, R)], sem_s[b]
        ).wait()


def kernel(token_ids, embedding_matrix):
    batch, seq = token_ids.shape
    n_rows, dim = embedding_matrix.shape
    assert seq == SEQ and batch % (NUM_WORKERS * R * 2) == 0
    ids = token_ids.astype(jnp.int32)

    mesh = plsc.VectorSubcoreMesh(
        core_axis_name="c", subcore_axis_name="s",
        num_cores=NUM_CORES, num_subcores=NUM_SUBCORES,
    )
    run = pl.kernel(
        _embed_lookup,
        out_type=jax.ShapeDtypeStruct((batch, seq, dim), jnp.float32),
        mesh=mesh,
        scratch_types=[
            pltpu.VMEM((2, R, SEQ), jnp.int32),
            pltpu.VMEM((2, R, SEQ, dim), jnp.float32),
            pltpu.SemaphoreType.DMA,
            pltpu.SemaphoreType.DMA,
            pltpu.SemaphoreType.DMA,
            pltpu.SemaphoreType.DMA,
            pltpu.SemaphoreType.DMA,
        ],
        compiler_params=pltpu.CompilerParams(use_tc_tiling_on_sc=False),
    )
    return run(ids, embedding_matrix)
